# Initial kernel scaffold; baseline (speedup 1.0000x reference)
import jax
import jax.numpy as jnp
from jax.experimental import pallas as pl


def kernel(x, edge_index, edge_weight, W, b):
    raise NotImplementedError("write your pallas kernel here")


# speedup vs baseline: 10.0846x; 10.0846x over previous
import functools

import jax
import jax.numpy as jnp
from jax import lax
from jax.experimental import pallas as pl
from jax.experimental.pallas import tpu as pltpu
from jax.experimental.pallas import tpu_sc as plsc

N = 10000
E = 320000
D = 128

NC = 2
NS = 16
NW = NC * NS
CHUNK = 128
CPW = 80
EP = NW * CPW * CHUNK
NP = 10240
RPT = NP // NS
IDXG = 8

_MESH = plsc.VectorSubcoreMesh(
    core_axis_name="c", subcore_axis_name="s", num_cores=NC, num_subcores=NS)


def _sc_deg_body(col_hbm, ew_hbm, out_hbm, colv, ewv, zv, deg_sh, sem):
    cid = lax.axis_index("c")
    sid = lax.axis_index("s")
    w = cid * NS + sid

    def z(i, carry):
        zv[pl.ds(i * 16, 16)] = jnp.zeros((16,), jnp.float32)
        return carry

    lax.fori_loop(0, RPT // 16, z, 0)
    pltpu.sync_copy(zv, deg_sh.at[pl.ds(sid * RPT, RPT)])
    plsc.subcore_barrier()

    pltpu.sync_copy(col_hbm.at[pl.ds(w * CPW, CPW)], colv)
    pltpu.sync_copy(ew_hbm.at[pl.ds(w * CPW, CPW)], ewv)
    K = 16
    for base in range(0, CPW, K):
        descs = [
            pltpu.async_copy(ewv.at[jc], deg_sh.at[colv.at[jc]], sem, add=True)
            for jc in range(base, min(base + K, CPW))
        ]
        for dsc in descs:
            dsc.wait()
    plsc.subcore_barrier()
    pltpu.sync_copy(deg_sh.at[pl.ds(sid * RPT, RPT)],
                    out_hbm.at[cid, pl.ds(sid * RPT, RPT)])


_deg_call = functools.partial(
    pl.kernel,
    out_type=jax.ShapeDtypeStruct((NC, NP), jnp.float32),
    mesh=_MESH,
    scratch_types=[
        pltpu.VMEM((CPW, CHUNK), jnp.int32),
        pltpu.VMEM((CPW, CHUNK), jnp.float32),
        pltpu.VMEM((RPT,), jnp.float32),
        pltpu.VMEM_SHARED((NP,), jnp.float32),
        pltpu.SemaphoreType.DMA,
    ],
)(_sc_deg_body)


def _sc_msg_body(g_hbm, row_hbm, col_hbm, ew_hbm, out_hbm,
                 rowv, colv, ewv, rv, mv, acc_sh):
    cid = lax.axis_index("c")
    sid = lax.axis_index("s")
    w = cid * NS + sid

    def z(i, carry):
        mv[i // 8, pl.ds((i % 8) * 16, 16)] = jnp.zeros((16,), jnp.float32)
        return carry

    lax.fori_loop(0, CHUNK * (D // 16), z, 0)
    for t in range(RPT // CHUNK):
        pltpu.sync_copy(mv, acc_sh.at[pl.ds(sid * RPT + t * CHUNK, CHUNK)])
    plsc.subcore_barrier()

    dnums = lax.GatherDimensionNumbers(
        offset_dims=(), collapsed_slice_dims=(0,), start_index_map=(0,))

    def group_body(og, carry):
        base = w * CPW + og * IDXG
        pltpu.sync_copy(row_hbm.at[pl.ds(base, IDXG)], rowv)
        pltpu.sync_copy(col_hbm.at[pl.ds(base, IDXG)], colv)
        pltpu.sync_copy(ew_hbm.at[pl.ds(base, IDXG)], ewv)

        def chunk_body(jc, c1):
            pltpu.sync_copy(g_hbm.at[rowv.at[jc]], rv)

            def grp(gi, c2):
                ewg = ewv[jc, pl.ds(gi * 16, 16)]
                for j in range(16):
                    bc = lax.gather(ewg, jnp.full((16, 1), j, jnp.int32),
                                    dnums, slice_sizes=(1,),
                                    mode=lax.GatherScatterMode.PROMISE_IN_BOUNDS)
                    for dk in range(D // 16):
                        sl = pl.ds(dk * 16, 16)
                        mv[gi * 16 + j, sl] = rv[gi * 16 + j, sl] * bc
                return c2

            lax.fori_loop(0, CHUNK // 16, grp, 0)
            pltpu.sync_copy(mv, acc_sh.at[colv.at[jc]], add=True)
            return c1

        lax.fori_loop(0, IDXG, chunk_body, 0)
        return carry

    lax.fori_loop(0, CPW // IDXG, group_body, 0)
    plsc.subcore_barrier()
    pltpu.sync_copy(acc_sh.at[pl.ds(sid * RPT, RPT)],
                    out_hbm.at[cid, pl.ds(sid * RPT, RPT)])


_msg_call = functools.partial(
    pl.kernel,
    out_type=jax.ShapeDtypeStruct((NC, NP, D), jnp.float32),
    mesh=_MESH,
    scratch_types=[
        pltpu.VMEM((IDXG, CHUNK), jnp.int32),
        pltpu.VMEM((IDXG, CHUNK), jnp.int32),
        pltpu.VMEM((IDXG, CHUNK), jnp.float32),
        pltpu.VMEM((CHUNK, D), jnp.float32),
        pltpu.VMEM((CHUNK, D), jnp.float32),
        pltpu.VMEM_SHARED((NP, D), jnp.float32),
    ],
)(_sc_msg_body)


def _tc_pre_body(x_ref, w_ref, d0_ref, d1_ref, b_ref, g_ref, base_ref, dinv_ref):
    h = jnp.dot(x_ref[...], w_ref[...], preferred_element_type=jnp.float32)
    deg = d0_ref[...] + d1_ref[...] + 1.0
    dinv = lax.rsqrt(deg)
    g_ref[...] = h * dinv[:, None]
    base_ref[...] = h * (dinv * dinv)[:, None] + b_ref[...][None, :]
    dinv_ref[...] = dinv


_BR = 256

def _tc_pre(xp, Wm, d0, d1, b):
    grid = (NP // _BR,)
    return pl.pallas_call(
        _tc_pre_body,
        grid=grid,
        in_specs=[
            pl.BlockSpec((_BR, D), lambda i: (i, 0)),
            pl.BlockSpec((D, D), lambda i: (0, 0)),
            pl.BlockSpec((_BR,), lambda i: (i,)),
            pl.BlockSpec((_BR,), lambda i: (i,)),
            pl.BlockSpec((D,), lambda i: (0,)),
        ],
        out_specs=[
            pl.BlockSpec((_BR, D), lambda i: (i, 0)),
            pl.BlockSpec((_BR, D), lambda i: (i, 0)),
            pl.BlockSpec((_BR,), lambda i: (i,)),
        ],
        out_shape=[
            jax.ShapeDtypeStruct((NP, D), jnp.float32),
            jax.ShapeDtypeStruct((NP, D), jnp.float32),
            jax.ShapeDtypeStruct((NP,), jnp.float32),
        ],
    )(xp, Wm, d0, d1, b)


def _tc_post_body(a0_ref, a1_ref, dinv_ref, base_ref, o_ref):
    o_ref[...] = ((a0_ref[...] + a1_ref[...]) * dinv_ref[...][:, None]
                  + base_ref[...])


def _tc_post(a0, a1, dinv, base):
    grid = (NP // _BR,)
    return pl.pallas_call(
        _tc_post_body,
        grid=grid,
        in_specs=[
            pl.BlockSpec((_BR, D), lambda i: (i, 0)),
            pl.BlockSpec((_BR, D), lambda i: (i, 0)),
            pl.BlockSpec((_BR,), lambda i: (i,)),
            pl.BlockSpec((_BR, D), lambda i: (i, 0)),
        ],
        out_specs=pl.BlockSpec((_BR, D), lambda i: (i, 0)),
        out_shape=jax.ShapeDtypeStruct((NP, D), jnp.float32),
    )(a0, a1, dinv, base)


def kernel(x, edge_index, edge_weight, W, b):
    row = edge_index[0]
    col = edge_index[1]
    pad_e = EP - E
    rowp = jnp.concatenate([row, jnp.zeros((pad_e,), row.dtype)]).reshape(-1, CHUNK)
    colp = jnp.concatenate([col, jnp.zeros((pad_e,), col.dtype)]).reshape(-1, CHUNK)
    ewp = jnp.concatenate(
        [edge_weight, jnp.zeros((pad_e,), edge_weight.dtype)]).reshape(-1, CHUNK)
    xp = jnp.concatenate([x, jnp.zeros((NP - N, D), x.dtype)])

    degp = _deg_call(colp, ewp)
    g, base, dinv = _tc_pre(xp, W, degp[0], degp[1], b)
    acc = _msg_call(g, rowp, colp, ewp)
    outp = _tc_post(acc[0], acc[1], dinv, base)
    return outp[:N]


# speedup vs baseline: 11.9750x; 1.1875x over previous
import functools

import jax
import jax.numpy as jnp
from jax import lax
from jax.experimental import pallas as pl
from jax.experimental.pallas import tpu as pltpu
from jax.experimental.pallas import tpu_sc as plsc

N = 10000
E = 320000
D = 128

NC = 2
NS = 16
NW = NC * NS
CHUNK = 128
CPW = 80
EP = NW * CPW * CHUNK
NP = 10240
RPT = NP // NS
CH2 = 64
SCPW = CPW * CHUNK // CH2
SPR = 32
NRND = SCPW // SPR

_MESH = plsc.VectorSubcoreMesh(
    core_axis_name="c", subcore_axis_name="s", num_cores=NC, num_subcores=NS)


def _sc_deg_body(col_hbm, ew_hbm, out_hbm, colv, ewv, zv, deg_sh, sem):
    cid = lax.axis_index("c")
    sid = lax.axis_index("s")
    w = cid * NS + sid

    def z(i, carry):
        zv[pl.ds(i * 16, 16)] = jnp.zeros((16,), jnp.float32)
        return carry

    lax.fori_loop(0, RPT // 16, z, 0)
    pltpu.sync_copy(zv, deg_sh.at[pl.ds(sid * RPT, RPT)])
    plsc.subcore_barrier()

    pltpu.sync_copy(col_hbm.at[pl.ds(w * SCPW, SCPW)], colv)
    pltpu.sync_copy(ew_hbm.at[pl.ds(w * SCPW, SCPW)], ewv)
    K = 16
    for base in range(0, SCPW, K):
        descs = [
            pltpu.async_copy(ewv.at[jc], deg_sh.at[colv.at[jc]], sem, add=True)
            for jc in range(base, min(base + K, SCPW))
        ]
        for dsc in descs:
            dsc.wait()
    plsc.subcore_barrier()
    pltpu.sync_copy(deg_sh.at[pl.ds(sid * RPT, RPT)],
                    out_hbm.at[cid, pl.ds(sid * RPT, RPT)])


_deg_call = functools.partial(
    pl.kernel,
    out_type=jax.ShapeDtypeStruct((NC, NP), jnp.float32),
    mesh=_MESH,
    scratch_types=[
        pltpu.VMEM((SCPW, CH2), jnp.int32),
        pltpu.VMEM((SCPW, CH2), jnp.float32),
        pltpu.VMEM((RPT,), jnp.float32),
        pltpu.VMEM_SHARED((NP,), jnp.float32),
        pltpu.SemaphoreType.DMA,
    ],
)(_sc_deg_body)


def _sc_msg_body(g_hbm, row_hbm, col_hbm, ew_hbm, out_hbm,
                 rowv, colv, ewv, rva, rvb, mva, mvb, acc_sh,
                 gsa, gsb, ssa, ssb):
    cid = lax.axis_index("c")
    sid = lax.axis_index("s")
    w = cid * NS + sid

    def z(i, carry):
        mva[i // (D // 16), pl.ds((i % (D // 16)) * 16, 16)] = (
            jnp.zeros((16,), jnp.float32))
        return carry

    lax.fori_loop(0, CH2 * (D // 16), z, 0)
    for t in range(RPT // CH2):
        pltpu.sync_copy(mva, acc_sh.at[pl.ds(sid * RPT + t * CH2, CH2)])
    plsc.subcore_barrier()

    dnums = lax.GatherDimensionNumbers(
        offset_dims=(), collapsed_slice_dims=(0,), start_index_map=(0,))

    def compute(rv, mv, jc):
        def grp(gi, c2):
            ewg = ewv[jc, pl.ds(gi * 16, 16)]
            for j in range(16):
                bc = lax.gather(ewg, jnp.full((16, 1), j, jnp.int32),
                                dnums, slice_sizes=(1,),
                                mode=lax.GatherScatterMode.PROMISE_IN_BOUNDS)
                for dk in range(D // 16):
                    sl = pl.ds(dk * 16, 16)
                    mv[gi * 16 + j, sl] = rv[gi * 16 + j, sl] * bc
            return c2

        lax.fori_loop(0, CH2 // 16, grp, 0)

    def round_body(rnd, carry):
        base = w * SCPW + rnd * SPR
        pltpu.sync_copy(row_hbm.at[pl.ds(base, SPR)], rowv)
        pltpu.sync_copy(col_hbm.at[pl.ds(base, SPR)], colv)
        pltpu.sync_copy(ew_hbm.at[pl.ds(base, SPR)], ewv)

        pltpu.async_copy(g_hbm.at[rowv.at[0]], rva, gsa)

        def pair(i, c1):
            ja = 2 * i
            jb = 2 * i + 1
            pltpu.make_async_copy(g_hbm.at[rowv.at[ja]], rva, gsa).wait()
            pltpu.async_copy(g_hbm.at[rowv.at[jb]], rvb, gsb)

            @pl.when(i > 0)
            def _():
                pltpu.make_async_copy(
                    mva, acc_sh.at[colv.at[ja]], ssa).wait()

            compute(rva, mva, ja)
            pltpu.async_copy(mva, acc_sh.at[colv.at[ja]], ssa, add=True)
            pltpu.make_async_copy(g_hbm.at[rowv.at[jb]], rvb, gsb).wait()

            @pl.when(i < SPR // 2 - 1)
            def _():
                pltpu.async_copy(g_hbm.at[rowv.at[ja + 2]], rva, gsa)

            @pl.when(i > 0)
            def _():
                pltpu.make_async_copy(
                    mvb, acc_sh.at[colv.at[jb]], ssb).wait()

            compute(rvb, mvb, jb)
            pltpu.async_copy(mvb, acc_sh.at[colv.at[jb]], ssb, add=True)
            return c1

        lax.fori_loop(0, SPR // 2, pair, 0)
        pltpu.make_async_copy(mva, acc_sh.at[colv.at[SPR - 2]], ssa).wait()
        pltpu.make_async_copy(mvb, acc_sh.at[colv.at[SPR - 1]], ssb).wait()
        return carry

    lax.fori_loop(0, NRND, round_body, 0)
    plsc.subcore_barrier()
    pltpu.sync_copy(acc_sh.at[pl.ds(sid * RPT, RPT)],
                    out_hbm.at[cid, pl.ds(sid * RPT, RPT)])


_msg_call = functools.partial(
    pl.kernel,
    out_type=jax.ShapeDtypeStruct((NC, NP, D), jnp.float32),
    mesh=_MESH,
    scratch_types=[
        pltpu.VMEM((SPR, CH2), jnp.int32),
        pltpu.VMEM((SPR, CH2), jnp.int32),
        pltpu.VMEM((SPR, CH2), jnp.float32),
        pltpu.VMEM((CH2, D), jnp.float32),
        pltpu.VMEM((CH2, D), jnp.float32),
        pltpu.VMEM((CH2, D), jnp.float32),
        pltpu.VMEM((CH2, D), jnp.float32),
        pltpu.VMEM_SHARED((NP, D), jnp.float32),
        pltpu.SemaphoreType.DMA,
        pltpu.SemaphoreType.DMA,
        pltpu.SemaphoreType.DMA,
        pltpu.SemaphoreType.DMA,
    ],
)(_sc_msg_body)


def _tc_pre_body(x_ref, w_ref, d0_ref, d1_ref, b_ref, g_ref, base_ref, dinv_ref):
    h = jnp.dot(x_ref[...], w_ref[...], preferred_element_type=jnp.float32)
    deg = d0_ref[...] + d1_ref[...] + 1.0
    dinv = lax.rsqrt(deg)
    g_ref[...] = h * dinv[:, None]
    base_ref[...] = h * (dinv * dinv)[:, None] + b_ref[...][None, :]
    dinv_ref[...] = dinv


_BR = 256

def _tc_pre(xp, Wm, d0, d1, b):
    grid = (NP // _BR,)
    return pl.pallas_call(
        _tc_pre_body,
        grid=grid,
        in_specs=[
            pl.BlockSpec((_BR, D), lambda i: (i, 0)),
            pl.BlockSpec((D, D), lambda i: (0, 0)),
            pl.BlockSpec((_BR,), lambda i: (i,)),
            pl.BlockSpec((_BR,), lambda i: (i,)),
            pl.BlockSpec((D,), lambda i: (0,)),
        ],
        out_specs=[
            pl.BlockSpec((_BR, D), lambda i: (i, 0)),
            pl.BlockSpec((_BR, D), lambda i: (i, 0)),
            pl.BlockSpec((_BR,), lambda i: (i,)),
        ],
        out_shape=[
            jax.ShapeDtypeStruct((NP, D), jnp.float32),
            jax.ShapeDtypeStruct((NP, D), jnp.float32),
            jax.ShapeDtypeStruct((NP,), jnp.float32),
        ],
    )(xp, Wm, d0, d1, b)


def _tc_post_body(a0_ref, a1_ref, dinv_ref, base_ref, o_ref):
    o_ref[...] = ((a0_ref[...] + a1_ref[...]) * dinv_ref[...][:, None]
                  + base_ref[...])


def _tc_post(a0, a1, dinv, base):
    grid = (NP // _BR,)
    return pl.pallas_call(
        _tc_post_body,
        grid=grid,
        in_specs=[
            pl.BlockSpec((_BR, D), lambda i: (i, 0)),
            pl.BlockSpec((_BR, D), lambda i: (i, 0)),
            pl.BlockSpec((_BR,), lambda i: (i,)),
            pl.BlockSpec((_BR, D), lambda i: (i, 0)),
        ],
        out_specs=pl.BlockSpec((_BR, D), lambda i: (i, 0)),
        out_shape=jax.ShapeDtypeStruct((NP, D), jnp.float32),
    )(a0, a1, dinv, base)


def kernel(x, edge_index, edge_weight, W, b):
    row = edge_index[0]
    col = edge_index[1]
    pad_e = EP - E
    rowp = jnp.concatenate([row, jnp.zeros((pad_e,), row.dtype)]).reshape(-1, CH2)
    colp = jnp.concatenate([col, jnp.zeros((pad_e,), col.dtype)]).reshape(-1, CH2)
    ewp = jnp.concatenate(
        [edge_weight, jnp.zeros((pad_e,), edge_weight.dtype)]).reshape(-1, CH2)
    xp = jnp.concatenate([x, jnp.zeros((NP - N, D), x.dtype)])

    degp = _deg_call(colp, ewp)
    g, base, dinv = _tc_pre(xp, W, degp[0], degp[1], b)
    acc = _msg_call(g, rowp, colp, ewp)
    outp = _tc_post(acc[0], acc[1], dinv, base)
    return outp[:N]


# speedup vs baseline: 14.6535x; 1.2237x over previous
import functools

import jax
import jax.numpy as jnp
from jax import lax
from jax.experimental import pallas as pl
from jax.experimental.pallas import tpu as pltpu
from jax.experimental.pallas import tpu_sc as plsc

N = 10000
E = 320000
D = 128

NC = 2
NS = 16
NW = NC * NS
CHUNK = 128
CPW = 80
EP = NW * CPW * CHUNK
NP = 10240
RPT = NP // NS
CH2 = 32
SUBW = CPW * CHUNK // CH2
SPR = 32
NRND = SUBW // SPR
NB = 8
MB = 2

_MESH = plsc.VectorSubcoreMesh(
    core_axis_name="c", subcore_axis_name="s", num_cores=NC, num_subcores=NS)


def _sc_deg_body(col_hbm, ew_hbm, out_hbm, colv, ewv, zv, deg_sh, sem):
    cid = lax.axis_index("c")
    sid = lax.axis_index("s")
    w = cid * NS + sid

    def z(i, carry):
        zv[pl.ds(i * 16, 16)] = jnp.zeros((16,), jnp.float32)
        return carry

    lax.fori_loop(0, RPT // 16, z, 0)
    pltpu.sync_copy(zv, deg_sh.at[pl.ds(sid * RPT, RPT)])
    plsc.subcore_barrier()

    pltpu.sync_copy(col_hbm.at[pl.ds(w * CPW, CPW)], colv)
    pltpu.sync_copy(ew_hbm.at[pl.ds(w * CPW, CPW)], ewv)
    K = 16
    for base in range(0, CPW, K):
        descs = [
            pltpu.async_copy(ewv.at[jc], deg_sh.at[colv.at[jc]], sem, add=True)
            for jc in range(base, min(base + K, CPW))
        ]
        for dsc in descs:
            dsc.wait()
    plsc.subcore_barrier()
    pltpu.sync_copy(deg_sh.at[pl.ds(sid * RPT, RPT)],
                    out_hbm.at[cid, pl.ds(sid * RPT, RPT)])


_deg_call = functools.partial(
    pl.kernel,
    out_type=jax.ShapeDtypeStruct((NC, NP), jnp.float32),
    mesh=_MESH,
    scratch_types=[
        pltpu.VMEM((CPW, CHUNK), jnp.int32),
        pltpu.VMEM((CPW, CHUNK), jnp.float32),
        pltpu.VMEM((RPT,), jnp.float32),
        pltpu.VMEM_SHARED((NP,), jnp.float32),
        pltpu.SemaphoreType.DMA,
    ],
)(_sc_deg_body)


def _sc_msg_body(g_hbm, row_hbm, col_hbm, ew_hbm, out_hbm,
                 rowv, colv, ewv,
                 rv0, rv1, rv2, rv3, rv4, rv5, rv6, rv7,
                 mv0, mv1, acc_sh,
                 gs0, gs1, gs2, gs3, gs4, gs5, gs6, gs7, ss0, ss1):
    rvs = [rv0, rv1, rv2, rv3, rv4, rv5, rv6, rv7]
    gss = [gs0, gs1, gs2, gs3, gs4, gs5, gs6, gs7]
    mvs = [mv0, mv1]
    sss = [ss0, ss1]
    cid = lax.axis_index("c")
    sid = lax.axis_index("s")
    w = cid * NS + sid

    def z(i, carry):
        mv0[i // (D // 16), pl.ds((i % (D // 16)) * 16, 16)] = (
            jnp.zeros((16,), jnp.float32))
        return carry

    lax.fori_loop(0, CH2 * (D // 16), z, 0)
    for t in range(RPT // CH2):
        pltpu.sync_copy(mv0, acc_sh.at[pl.ds(sid * RPT + t * CH2, CH2)])
    plsc.subcore_barrier()

    dnums = lax.GatherDimensionNumbers(
        offset_dims=(), collapsed_slice_dims=(0,), start_index_map=(0,))

    def compute(rv, mv, jc):
        def grp(gi, c2):
            ewg = ewv[pl.ds(jc * CH2 + gi * 16, 16)]
            for j in range(16):
                bc = lax.gather(ewg, jnp.full((16, 1), j, jnp.int32),
                                dnums, slice_sizes=(1,),
                                mode=lax.GatherScatterMode.PROMISE_IN_BOUNDS)
                for dk in range(D // 16):
                    sl = pl.ds(dk * 16, 16)
                    mv[gi * 16 + j, sl] = rv[gi * 16 + j, sl] * bc
            return c2

        lax.fori_loop(0, CH2 // 16, grp, 0)

    def gslice(j):
        return rowv.at[pl.ds(j * CH2, CH2)]

    def round_body(rnd, carry):
        base = w * SUBW + rnd * SPR
        pltpu.sync_copy(row_hbm.at[pl.ds(base * CH2, SPR * CH2)], rowv)
        pltpu.sync_copy(col_hbm.at[pl.ds(base, SPR)], colv)
        pltpu.sync_copy(ew_hbm.at[pl.ds(base * CH2, SPR * CH2)], ewv)

        for pb in range(NB - 1):
            pltpu.async_copy(g_hbm.at[gslice(pb)], rvs[pb], gss[pb])

        def kstep(k, c1):
            for b in range(NB):
                j = k * NB + b
                pltpu.make_async_copy(
                    g_hbm.at[gslice(j)], rvs[b], gss[b]).wait()
                jn = j + NB - 1
                bn = (b + NB - 1) % NB

                @pl.when(jn < SPR)
                def _(jn=jn, bn=bn):
                    pltpu.async_copy(g_hbm.at[gslice(jn)], rvs[bn], gss[bn])

                m = b % MB
                if b >= MB:
                    pltpu.make_async_copy(
                        mvs[m], acc_sh.at[colv.at[j]], sss[m]).wait()
                else:
                    @pl.when(k > 0)
                    def _(m=m, j=j):
                        pltpu.make_async_copy(
                            mvs[m], acc_sh.at[colv.at[j]], sss[m]).wait()

                compute(rvs[b], mvs[m], j)
                pltpu.async_copy(mvs[m], acc_sh.at[colv.at[j]], sss[m],
                                 add=True)
            return c1

        lax.fori_loop(0, SPR // NB, kstep, 0)
        pltpu.make_async_copy(mv0, acc_sh.at[colv.at[SPR - 2]], ss0).wait()
        pltpu.make_async_copy(mv1, acc_sh.at[colv.at[SPR - 1]], ss1).wait()
        return carry

    lax.fori_loop(0, NRND, round_body, 0)
    plsc.subcore_barrier()
    pltpu.sync_copy(acc_sh.at[pl.ds(sid * RPT, RPT)],
                    out_hbm.at[cid, pl.ds(sid * RPT, RPT)])


_msg_call = functools.partial(
    pl.kernel,
    out_type=jax.ShapeDtypeStruct((NC, NP, D), jnp.float32),
    mesh=_MESH,
    scratch_types=(
        [
            pltpu.VMEM((SPR * CH2,), jnp.int32),
            pltpu.VMEM((SPR, CH2), jnp.int32),
            pltpu.VMEM((SPR * CH2,), jnp.float32),
        ]
        + [pltpu.VMEM((CH2, D), jnp.float32) for _ in range(NB)]
        + [pltpu.VMEM((CH2, D), jnp.float32) for _ in range(MB)]
        + [pltpu.VMEM_SHARED((NP, D), jnp.float32)]
        + [pltpu.SemaphoreType.DMA for _ in range(NB + MB)]
    ),
)(_sc_msg_body)


def _tc_pre_body(x_ref, w_ref, d0_ref, d1_ref, b_ref, g_ref, base_ref, dinv_ref):
    h = jnp.dot(x_ref[...], w_ref[...], preferred_element_type=jnp.float32)
    deg = d0_ref[...] + d1_ref[...] + 1.0
    dinv = lax.rsqrt(deg)
    g_ref[...] = h * dinv[:, None]
    base_ref[...] = h * (dinv * dinv)[:, None] + b_ref[...][None, :]
    dinv_ref[...] = dinv


_BR = 256

def _tc_pre(xp, Wm, d0, d1, b):
    grid = (NP // _BR,)
    return pl.pallas_call(
        _tc_pre_body,
        grid=grid,
        in_specs=[
            pl.BlockSpec((_BR, D), lambda i: (i, 0)),
            pl.BlockSpec((D, D), lambda i: (0, 0)),
            pl.BlockSpec((_BR,), lambda i: (i,)),
            pl.BlockSpec((_BR,), lambda i: (i,)),
            pl.BlockSpec((D,), lambda i: (0,)),
        ],
        out_specs=[
            pl.BlockSpec((_BR, D), lambda i: (i, 0)),
            pl.BlockSpec((_BR, D), lambda i: (i, 0)),
            pl.BlockSpec((_BR,), lambda i: (i,)),
        ],
        out_shape=[
            jax.ShapeDtypeStruct((NP, D), jnp.float32),
            jax.ShapeDtypeStruct((NP, D), jnp.float32),
            jax.ShapeDtypeStruct((NP,), jnp.float32),
        ],
    )(xp, Wm, d0, d1, b)


def _tc_post_body(a0_ref, a1_ref, dinv_ref, base_ref, o_ref):
    o_ref[...] = ((a0_ref[...] + a1_ref[...]) * dinv_ref[...][:, None]
                  + base_ref[...])


def _tc_post(a0, a1, dinv, base):
    grid = (NP // _BR,)
    return pl.pallas_call(
        _tc_post_body,
        grid=grid,
        in_specs=[
            pl.BlockSpec((_BR, D), lambda i: (i, 0)),
            pl.BlockSpec((_BR, D), lambda i: (i, 0)),
            pl.BlockSpec((_BR,), lambda i: (i,)),
            pl.BlockSpec((_BR, D), lambda i: (i, 0)),
        ],
        out_specs=pl.BlockSpec((_BR, D), lambda i: (i, 0)),
        out_shape=jax.ShapeDtypeStruct((NP, D), jnp.float32),
    )(a0, a1, dinv, base)


def kernel(x, edge_index, edge_weight, W, b):
    row = edge_index[0]
    col = edge_index[1]
    pad_e = EP - E
    rowp = jnp.concatenate([row, jnp.zeros((pad_e,), row.dtype)])
    colp = jnp.concatenate([col, jnp.zeros((pad_e,), col.dtype)])
    ewp = jnp.concatenate([edge_weight, jnp.zeros((pad_e,), edge_weight.dtype)])
    xp = jnp.concatenate([x, jnp.zeros((NP - N, D), x.dtype)])

    degp = _deg_call(colp.reshape(-1, CHUNK),
                     ewp.reshape(-1, CHUNK))
    g, base, dinv = _tc_pre(xp, W, degp[0], degp[1], b)
    acc = _msg_call(g, rowp, colp.reshape(-1, CH2),
                    ewp)
    outp = _tc_post(acc[0], acc[1], dinv, base)
    return outp[:N]


# speedup vs baseline: 21.1045x; 1.4402x over previous
import functools

import jax
import jax.numpy as jnp
from jax import lax
from jax.experimental import pallas as pl
from jax.experimental.pallas import tpu as pltpu
from jax.experimental.pallas import tpu_sc as plsc

N = 10000
E = 320000
D = 128

NC = 2
NS = 16
NW = NC * NS
CHUNK = 128
CPW = 80
EP = NW * CPW * CHUNK
NP = 10112
RPT = NP // NS
CH2 = 16
SUBW = CPW * CHUNK // CH2
SPR = 64
NRND = SUBW // SPR
NB = 4

NG = 5000
NA = 10000

_MESH = plsc.VectorSubcoreMesh(
    core_axis_name="c", subcore_axis_name="s", num_cores=NC, num_subcores=NS)


NPD = 10240
RPD = NPD // NS


def _sc_deg_body(col_hbm, ew_hbm, out_hbm, colv, ewv, zv, deg_sh, sem):
    cid = lax.axis_index("c")
    sid = lax.axis_index("s")
    w = cid * NS + sid

    def z(i, carry):
        zv[pl.ds(i * 16, 16)] = jnp.zeros((16,), jnp.float32)
        return carry

    lax.fori_loop(0, RPD // 16, z, 0)
    pltpu.sync_copy(zv, deg_sh.at[pl.ds(sid * RPD, RPD)])
    plsc.subcore_barrier()

    pltpu.sync_copy(col_hbm.at[pl.ds(w * CPW, CPW)], colv)
    pltpu.sync_copy(ew_hbm.at[pl.ds(w * CPW, CPW)], ewv)
    K = 16
    for base in range(0, CPW, K):
        descs = [
            pltpu.async_copy(ewv.at[jc], deg_sh.at[colv.at[jc]], sem, add=True)
            for jc in range(base, min(base + K, CPW))
        ]
        for dsc in descs:
            dsc.wait()
    plsc.subcore_barrier()
    pltpu.sync_copy(deg_sh.at[pl.ds(sid * RPD, RPD)],
                    out_hbm.at[cid, pl.ds(sid * RPD, RPD)])


_deg_call = functools.partial(
    pl.kernel,
    out_type=jax.ShapeDtypeStruct((NC, NPD), jnp.float32),
    mesh=_MESH,
    scratch_types=[
        pltpu.VMEM((CPW, CHUNK), jnp.int32),
        pltpu.VMEM((CPW, CHUNK), jnp.float32),
        pltpu.VMEM((RPD,), jnp.float32),
        pltpu.VMEM_SHARED((NPD,), jnp.float32),
        pltpu.SemaphoreType.DMA,
    ],
)(_sc_deg_body)


def _sc_msg_body(g_hbm, row_hbm, col_hbm, ew_hbm, out_hbm,
                 rowv, colv, ews,
                 rv0, rv1, rv2, rv3, g_sh, acc_sh,
                 gs0, gs1, gs2, gs3, ss0, ss1, ss2, ss3):
    rvs = [rv0, rv1, rv2, rv3]
    gss = [gs0, gs1, gs2, gs3]
    sss = [ss0, ss1, ss2, ss3]
    cid = lax.axis_index("c")
    sid = lax.axis_index("s")
    w = cid * NS + sid

    def z(i, carry):
        rv0[i // (D // 16), pl.ds((i % (D // 16)) * 16, 16)] = (
            jnp.zeros((16,), jnp.float32))
        return carry

    lax.fori_loop(0, CH2 * (D // 16), z, 0)

    @pl.when(sid < NS - 1)
    def _():
        for t in range(632 // CH2):
            pltpu.sync_copy(rv0, acc_sh.at[pl.ds(sid * 632 + t * CH2, CH2)])
        pltpu.sync_copy(rv0.at[pl.ds(0, 8)],
                        acc_sh.at[pl.ds(sid * 632 + 624, 8)])

    @pl.when(sid == NS - 1)
    def _():
        for t in range(520 // CH2):
            pltpu.sync_copy(rv0, acc_sh.at[pl.ds(15 * 632 + t * CH2, CH2)])
        pltpu.sync_copy(rv0.at[pl.ds(0, 8)],
                        acc_sh.at[pl.ds(15 * 632 + 512, 8)])

    @pl.when(sid < NS - 1)
    def _():
        pltpu.sync_copy(g_hbm.at[pl.ds(sid * 312, 312)],
                        g_sh.at[pl.ds(sid * 312, 312)])

    @pl.when(sid == NS - 1)
    def _():
        pltpu.sync_copy(g_hbm.at[pl.ds(15 * 312, NG - 15 * 312)],
                        g_sh.at[pl.ds(15 * 312, NG - 15 * 312)])

    plsc.subcore_barrier()

    himask = jnp.full((16,), -65536, jnp.int32)
    dnums = lax.GatherDimensionNumbers(
        offset_dims=(), collapsed_slice_dims=(0,), start_index_map=(0,))

    def compute(rv, jc):
        ewg = ews[pl.ds((jc % 32) * CH2, CH2)]

        def edge(e, c2):
            sw = lax.gather(ewg, jnp.full((16, 1), e, jnp.int32),
                            dnums, slice_sizes=(1,),
                            mode=lax.GatherScatterMode.PROMISE_IN_BOUNDS)
            bp = jnp.maximum(sw, 0.0)
            bn = jnp.maximum(-sw, 0.0)
            for c in range(D // 16):
                xi = plsc.bitcast(rv[e, pl.ds(16 * c, 16)], jnp.int32)
                lo = plsc.bitcast(xi << 16, jnp.float32)
                hi = plsc.bitcast(xi & himask, jnp.float32)
                rv[e, pl.ds(16 * c, 16)] = lo * bp + hi * bn
            return c2

        lax.fori_loop(0, CH2, edge, 0)

    def gslice(j):
        return rowv.at[pl.ds(j * CH2, CH2)]

    def cslice(j):
        return colv.at[j // 8, pl.ds((j % 8) * CH2, CH2)]

    def round_body(rnd, carry):
        base = w * SUBW + rnd * SPR
        pltpu.sync_copy(row_hbm.at[pl.ds(base * CH2, SPR * CH2)], rowv)
        pltpu.sync_copy(col_hbm.at[pl.ds(w * CPW + rnd * 8, 8)], colv)

        for pb in range(NB - 2):
            pltpu.async_copy(g_sh.at[gslice(pb)], rvs[pb], gss[pb])

        def kstep(k, c1):
            @pl.when(k % 8 == 0)
            def _():
                pltpu.sync_copy(
                    ew_hbm.at[pl.ds((base + k * NB) * CH2, 512)], ews)

            for b in range(NB):
                j = k * NB + b
                pltpu.make_async_copy(
                    g_sh.at[gslice(j)], rvs[b], gss[b]).wait()
                compute(rvs[b], j)
                pltpu.async_copy(rvs[b], acc_sh.at[cslice(j)], sss[b],
                                 add=True)
                jn = j + NB - 2
                bn = (b + NB - 2) % NB

                @pl.when(jn < SPR)
                def _(jn=jn, bn=bn, b=b, k=k):
                    if b < 2:
                        @pl.when(k > 0)
                        def _():
                            pltpu.make_async_copy(
                                rvs[bn], acc_sh.at[cslice(jn - NB)],
                                sss[bn]).wait()
                    else:
                        pltpu.make_async_copy(
                            rvs[bn], acc_sh.at[cslice(jn - NB)],
                            sss[bn]).wait()
                    pltpu.async_copy(g_sh.at[gslice(jn)], rvs[bn], gss[bn])
            return c1

        lax.fori_loop(0, SPR // NB, kstep, 0)
        for bb in range(NB):
            pltpu.make_async_copy(
                rvs[bb], acc_sh.at[cslice(SPR - NB + bb)], sss[bb]).wait()
        return carry

    lax.fori_loop(0, NRND, round_body, 0)
    plsc.subcore_barrier()

    @pl.when(sid < NS - 1)
    def _():
        pltpu.sync_copy(acc_sh.at[pl.ds(sid * 632, 632)],
                        out_hbm.at[cid, pl.ds(sid * 632, 632)])

    @pl.when(sid == NS - 1)
    def _():
        pltpu.sync_copy(acc_sh.at[pl.ds(15 * 632, NA - 15 * 632)],
                        out_hbm.at[cid, pl.ds(15 * 632, NA - 15 * 632)])


_msg_call = functools.partial(
    pl.kernel,
    out_type=jax.ShapeDtypeStruct((NC, NA, D), jnp.float32),
    mesh=_MESH,
    scratch_types=(
        [
            pltpu.VMEM((SPR * CH2,), jnp.int32),
            pltpu.VMEM((8, CHUNK), jnp.int32),
            pltpu.VMEM((512,), jnp.float32),
        ]
        + [pltpu.VMEM((CH2, D), jnp.float32) for _ in range(NB)]
        + [pltpu.VMEM_SHARED((NG, D), jnp.float32),
           pltpu.VMEM_SHARED((NA, D), jnp.float32)]
        + [pltpu.SemaphoreType.DMA for _ in range(2 * NB)]
    ),
    compiler_params=pltpu.CompilerParams(needs_layout_passes=False),
)(_sc_msg_body)


def _tc_pre_body(x_ref, w_ref, d0_ref, d1_ref, b_ref, g_ref, base_ref, dinv_ref):
    h = jnp.dot(x_ref[...], w_ref[...], preferred_element_type=jnp.float32)
    deg = d0_ref[...] + d1_ref[...] + 1.0
    dinv = lax.rsqrt(deg)
    g_ref[...] = h * dinv[:, None]
    base_ref[...] = h * (dinv * dinv)[:, None] + b_ref[...][None, :]
    dinv_ref[...] = dinv


_BR = 128

def _tc_pre(xp, Wm, d0, d1, b):
    grid = (NP // _BR,)
    return pl.pallas_call(
        _tc_pre_body,
        grid=grid,
        in_specs=[
            pl.BlockSpec((_BR, D), lambda i: (i, 0)),
            pl.BlockSpec((D, D), lambda i: (0, 0)),
            pl.BlockSpec((_BR,), lambda i: (i,)),
            pl.BlockSpec((_BR,), lambda i: (i,)),
            pl.BlockSpec((D,), lambda i: (0,)),
        ],
        out_specs=[
            pl.BlockSpec((_BR, D), lambda i: (i, 0)),
            pl.BlockSpec((_BR, D), lambda i: (i, 0)),
            pl.BlockSpec((_BR,), lambda i: (i,)),
        ],
        out_shape=[
            jax.ShapeDtypeStruct((NP, D), jnp.float32),
            jax.ShapeDtypeStruct((NP, D), jnp.float32),
            jax.ShapeDtypeStruct((NP,), jnp.float32),
        ],
    )(xp, Wm, d0, d1, b)


def _tc_post_body(a0_ref, a1_ref, dinv_ref, base_ref, o_ref):
    o_ref[...] = ((a0_ref[...] + a1_ref[...]) * dinv_ref[...]
                  + base_ref[...])


_BP = 80

def _tc_post(a0, a1, dinv, base):
    grid = (NA // _BP,)
    return pl.pallas_call(
        _tc_post_body,
        grid=grid,
        in_specs=[
            pl.BlockSpec((_BP, D), lambda i: (i, 0)),
            pl.BlockSpec((_BP, D), lambda i: (i, 0)),
            pl.BlockSpec((_BP, 1), lambda i: (i, 0)),
            pl.BlockSpec((_BP, D), lambda i: (i, 0)),
        ],
        out_specs=pl.BlockSpec((_BP, D), lambda i: (i, 0)),
        out_shape=jax.ShapeDtypeStruct((NA, D), jnp.float32),
    )(a0, a1, dinv[:, None], base)


def kernel(x, edge_index, edge_weight, W, b):
    row = edge_index[0]
    col = edge_index[1]
    pad_e = EP - E
    rowp = jnp.concatenate([row, jnp.zeros((pad_e,), row.dtype)])
    colp = jnp.concatenate([col, jnp.zeros((pad_e,), col.dtype)])
    ewp = jnp.concatenate([edge_weight, jnp.zeros((pad_e,), edge_weight.dtype)])
    xp = jnp.concatenate([x, jnp.zeros((NP - N, D), x.dtype)])

    degp = _deg_call(colp.reshape(-1, CHUNK),
                     ewp.reshape(-1, CHUNK))
    g, base, dinv = _tc_pre(xp, W, degp[0, :NP], degp[1, :NP], b)
    gb = g[:NA].astype(jnp.bfloat16)
    g2 = lax.bitcast_convert_type(
        gb.reshape(NG, 2, D).transpose(0, 2, 1), jnp.float32)
    rh = rowp >> 1
    ewsgn = ewp * (1.0 - 2.0 * (rowp & 1).astype(jnp.float32))
    acc = _msg_call(g2, rh, colp.reshape(-1, CHUNK),
                    ewsgn)
    outp = _tc_post(acc[0], acc[1], dinv[:NA], base[:NA])
    return outp


# speedup vs baseline: 22.7493x; 1.0779x over previous
import functools

import jax
import jax.numpy as jnp
from jax import lax
from jax.experimental import pallas as pl
from jax.experimental.pallas import tpu as pltpu
from jax.experimental.pallas import tpu_sc as plsc

N = 10000
E = 320000
D = 128

NC = 2
NS = 16
NW = NC * NS
CHUNK = 128
CPW = 80
EP = NW * CPW * CHUNK
NP = 10112
RPT = NP // NS
CH2 = 16
SUBW = CPW * CHUNK // CH2
SPR = 64
NRND = SUBW // SPR
NB = 4

NG = 5000
NA = 10000

_MESH = plsc.VectorSubcoreMesh(
    core_axis_name="c", subcore_axis_name="s", num_cores=NC, num_subcores=NS)


NPD = 10240
RPD = NPD // NS


def _sc_deg_body(col_hbm, ew_hbm, out_hbm, colv, ewv, zv, deg_sh, sem):
    cid = lax.axis_index("c")
    sid = lax.axis_index("s")
    w = cid * NS + sid

    def z(i, carry):
        zv[pl.ds(i * 16, 16)] = jnp.zeros((16,), jnp.float32)
        return carry

    lax.fori_loop(0, RPD // 16, z, 0)
    pltpu.sync_copy(zv, deg_sh.at[pl.ds(sid * RPD, RPD)])
    plsc.subcore_barrier()

    pltpu.sync_copy(col_hbm.at[pl.ds(w * CPW, CPW)], colv)
    pltpu.sync_copy(ew_hbm.at[pl.ds(w * CPW, CPW)], ewv)
    K = 16
    for base in range(0, CPW, K):
        descs = [
            pltpu.async_copy(ewv.at[jc], deg_sh.at[colv.at[jc]], sem, add=True)
            for jc in range(base, min(base + K, CPW))
        ]
        for dsc in descs:
            dsc.wait()
    plsc.subcore_barrier()
    pltpu.sync_copy(deg_sh.at[pl.ds(sid * RPD, RPD)],
                    out_hbm.at[cid, pl.ds(sid * RPD, RPD)])


_deg_call = functools.partial(
    pl.kernel,
    out_type=jax.ShapeDtypeStruct((NC, NPD), jnp.float32),
    mesh=_MESH,
    scratch_types=[
        pltpu.VMEM((CPW, CHUNK), jnp.int32),
        pltpu.VMEM((CPW, CHUNK), jnp.float32),
        pltpu.VMEM((RPD,), jnp.float32),
        pltpu.VMEM_SHARED((NPD,), jnp.float32),
        pltpu.SemaphoreType.DMA,
    ],
)(_sc_deg_body)


def _sc_msg_body(g_hbm, row_hbm, col_hbm, ew_hbm, out_hbm,
                 rowv, colv, ews,
                 rv0, rv1, rv2, rv3, g_sh, acc_sh,
                 gs0, gs1, gs2, gs3, ss0, ss1, ss2, ss3):
    rvs = [rv0, rv1, rv2, rv3]
    gss = [gs0, gs1, gs2, gs3]
    sss = [ss0, ss1, ss2, ss3]
    cid = lax.axis_index("c")
    sid = lax.axis_index("s")
    w = cid * NS + sid

    def z(i, carry):
        rv0[i // (D // 16), pl.ds((i % (D // 16)) * 16, 16)] = (
            jnp.zeros((16,), jnp.float32))
        return carry

    lax.fori_loop(0, CH2 * (D // 16), z, 0)

    @pl.when(sid < NS - 1)
    def _():
        for t in range(632 // CH2):
            pltpu.sync_copy(rv0, acc_sh.at[pl.ds(sid * 632 + t * CH2, CH2)])
        pltpu.sync_copy(rv0.at[pl.ds(0, 8)],
                        acc_sh.at[pl.ds(sid * 632 + 624, 8)])

    @pl.when(sid == NS - 1)
    def _():
        for t in range(520 // CH2):
            pltpu.sync_copy(rv0, acc_sh.at[pl.ds(15 * 632 + t * CH2, CH2)])
        pltpu.sync_copy(rv0.at[pl.ds(0, 8)],
                        acc_sh.at[pl.ds(15 * 632 + 512, 8)])

    @pl.when(sid < NS - 1)
    def _():
        pltpu.sync_copy(g_hbm.at[pl.ds(sid * 312, 312)],
                        g_sh.at[pl.ds(sid * 312, 312)])

    @pl.when(sid == NS - 1)
    def _():
        pltpu.sync_copy(g_hbm.at[pl.ds(15 * 312, NG - 15 * 312)],
                        g_sh.at[pl.ds(15 * 312, NG - 15 * 312)])

    plsc.subcore_barrier()

    himask = jnp.full((16,), -65536, jnp.int32)
    dnums = lax.GatherDimensionNumbers(
        offset_dims=(), collapsed_slice_dims=(0,), start_index_map=(0,))

    def compute(rv, jc):
        ewg = ews[pl.ds((jc % 32) * CH2, CH2)]

        def edge(e, c2):
            sw = lax.gather(ewg, jnp.full((16, 1), e, jnp.int32),
                            dnums, slice_sizes=(1,),
                            mode=lax.GatherScatterMode.PROMISE_IN_BOUNDS)
            bp = jnp.maximum(sw, 0.0)
            bn = jnp.maximum(-sw, 0.0)
            for c in range(D // 16):
                xi = plsc.bitcast(rv[e, pl.ds(16 * c, 16)], jnp.int32)
                lo = plsc.bitcast(xi << 16, jnp.float32)
                hi = plsc.bitcast(xi & himask, jnp.float32)
                rv[e, pl.ds(16 * c, 16)] = lo * bp + hi * bn
            return c2

        lax.fori_loop(0, CH2, edge, 0)

    def gslice(j):
        return rowv.at[pl.ds(j * CH2, CH2)]

    def cslice(j):
        return colv.at[j // 8, pl.ds((j % 8) * CH2, CH2)]

    def round_body(rnd, carry):
        base = w * SUBW + rnd * SPR
        pltpu.sync_copy(row_hbm.at[pl.ds(base * CH2, SPR * CH2)], rowv)
        pltpu.sync_copy(col_hbm.at[pl.ds(w * CPW + rnd * 8, 8)], colv)

        for pb in range(NB - 2):
            pltpu.async_copy(g_sh.at[gslice(pb)], rvs[pb], gss[pb])

        def kstep(k, c1):
            @pl.when(k % 8 == 0)
            def _():
                pltpu.sync_copy(
                    ew_hbm.at[pl.ds((base + k * NB) * CH2, 512)], ews)

            for b in range(NB):
                j = k * NB + b
                pltpu.make_async_copy(
                    g_sh.at[gslice(j)], rvs[b], gss[b]).wait()
                compute(rvs[b], j)
                pltpu.async_copy(rvs[b], acc_sh.at[cslice(j)], sss[b],
                                 add=True)
                jn = j + NB - 2
                bn = (b + NB - 2) % NB

                @pl.when(jn < SPR)
                def _(jn=jn, bn=bn, b=b, k=k):
                    if b < 2:
                        @pl.when(k > 0)
                        def _():
                            pltpu.make_async_copy(
                                rvs[bn], acc_sh.at[cslice(jn - NB)],
                                sss[bn]).wait()
                    else:
                        pltpu.make_async_copy(
                            rvs[bn], acc_sh.at[cslice(jn - NB)],
                            sss[bn]).wait()
                    pltpu.async_copy(g_sh.at[gslice(jn)], rvs[bn], gss[bn])
            return c1

        lax.fori_loop(0, SPR // NB, kstep, 0)
        for bb in range(NB):
            pltpu.make_async_copy(
                rvs[bb], acc_sh.at[cslice(SPR - NB + bb)], sss[bb]).wait()
        return carry

    lax.fori_loop(0, NRND, round_body, 0)
    plsc.subcore_barrier()

    @pl.when(sid < NS - 1)
    def _():
        pltpu.sync_copy(acc_sh.at[pl.ds(sid * 632, 632)],
                        out_hbm.at[cid, pl.ds(sid * 632, 632)])

    @pl.when(sid == NS - 1)
    def _():
        pltpu.sync_copy(acc_sh.at[pl.ds(15 * 632, NA - 15 * 632)],
                        out_hbm.at[cid, pl.ds(15 * 632, NA - 15 * 632)])


_msg_call = functools.partial(
    pl.kernel,
    out_type=jax.ShapeDtypeStruct((NC, NA, D), jnp.float32),
    mesh=_MESH,
    scratch_types=(
        [
            pltpu.VMEM((SPR * CH2,), jnp.int32),
            pltpu.VMEM((8, CHUNK), jnp.int32),
            pltpu.VMEM((512,), jnp.float32),
        ]
        + [pltpu.VMEM((CH2, D), jnp.float32) for _ in range(NB)]
        + [pltpu.VMEM_SHARED((NG, D), jnp.float32),
           pltpu.VMEM_SHARED((NA, D), jnp.float32)]
        + [pltpu.SemaphoreType.DMA for _ in range(2 * NB)]
    ),
    compiler_params=pltpu.CompilerParams(needs_layout_passes=False),
)(_sc_msg_body)


def _tc_pre_body(x_ref, w_ref, d0_ref, d1_ref, b_ref,
                 g2_ref, base_ref, dinv_ref):
    h = jnp.dot(x_ref[...], w_ref[...], preferred_element_type=jnp.float32)
    deg = d0_ref[...] + d1_ref[...] + 1.0
    dinv = lax.rsqrt(deg)
    g = h * dinv[:, None]
    base_ref[...] = h * (dinv * dinv)[:, None] + b_ref[...][None, :]
    dinv_ref[...] = dinv
    gb = lax.bitcast_convert_type(g, jnp.int32)
    gr = (gb + 0x8000) & jnp.int32(-65536)
    ge = gr.reshape(_BR // 2, 2, D)
    word = jnp.bitwise_or(lax.shift_right_logical(ge[:, 0, :], 16),
                          ge[:, 1, :])
    g2_ref[...] = lax.bitcast_convert_type(word, jnp.float32)


_BR = 128

def _tc_pre(xp, Wm, d0, d1, b):
    grid = (NP // _BR,)
    return pl.pallas_call(
        _tc_pre_body,
        grid=grid,
        in_specs=[
            pl.BlockSpec((_BR, D), lambda i: (i, 0)),
            pl.BlockSpec((D, D), lambda i: (0, 0)),
            pl.BlockSpec((_BR,), lambda i: (i,)),
            pl.BlockSpec((_BR,), lambda i: (i,)),
            pl.BlockSpec((D,), lambda i: (0,)),
        ],
        out_specs=[
            pl.BlockSpec((_BR // 2, D), lambda i: (i, 0)),
            pl.BlockSpec((_BR, D), lambda i: (i, 0)),
            pl.BlockSpec((_BR,), lambda i: (i,)),
        ],
        out_shape=[
            jax.ShapeDtypeStruct((NP // 2, D), jnp.float32),
            jax.ShapeDtypeStruct((NP, D), jnp.float32),
            jax.ShapeDtypeStruct((NP,), jnp.float32),
        ],
    )(xp, Wm, d0, d1, b)


def _tc_post_body(a0_ref, a1_ref, dinv_ref, base_ref, o_ref):
    o_ref[...] = ((a0_ref[...] + a1_ref[...]) * dinv_ref[...]
                  + base_ref[...])


_BP = 80

def _tc_post(a0, a1, dinv, base):
    grid = (NA // _BP,)
    return pl.pallas_call(
        _tc_post_body,
        grid=grid,
        in_specs=[
            pl.BlockSpec((_BP, D), lambda i: (i, 0)),
            pl.BlockSpec((_BP, D), lambda i: (i, 0)),
            pl.BlockSpec((_BP, 1), lambda i: (i, 0)),
            pl.BlockSpec((_BP, D), lambda i: (i, 0)),
        ],
        out_specs=pl.BlockSpec((_BP, D), lambda i: (i, 0)),
        out_shape=jax.ShapeDtypeStruct((NA, D), jnp.float32),
    )(a0, a1, dinv[:, None], base)


def kernel(x, edge_index, edge_weight, W, b):
    row = edge_index[0]
    col = edge_index[1]
    pad_e = EP - E
    rowp = jnp.concatenate([row, jnp.zeros((pad_e,), row.dtype)])
    colp = jnp.concatenate([col, jnp.zeros((pad_e,), col.dtype)])
    ewp = jnp.concatenate([edge_weight, jnp.zeros((pad_e,), edge_weight.dtype)])
    xp = jnp.concatenate([x, jnp.zeros((NP - N, D), x.dtype)])

    degp = _deg_call(colp.reshape(-1, CHUNK),
                     ewp.reshape(-1, CHUNK))
    g2f, base, dinv = _tc_pre(xp, W, degp[0, :NP], degp[1, :NP], b)
    g2 = g2f[:NG]
    rh = rowp >> 1
    ewsgn = ewp * (1.0 - 2.0 * (rowp & 1).astype(jnp.float32))
    acc = _msg_call(g2, rh, colp.reshape(-1, CHUNK),
                    ewsgn)
    outp = _tc_post(acc[0], acc[1], dinv[:NA], base[:NA])
    return outp
